# baseline (speedup 1.0000x reference)
import functools

import jax
import jax.numpy as jnp
from jax.experimental import pallas as pl
from jax.experimental.pallas import tpu as pltpu

_B, _T, _C, _NH = 1, 2048, 768, 32
_HD = _C // _NH
_BQ = 256
_SCALE = 1.0 / (_HD ** 0.5)


def _matmul_kernel(x_ref, w_ref, o_ref):
    o_ref[...] = jax.lax.dot_general(
        x_ref[...], w_ref[...],
        dimension_numbers=(((1,), (1,)), ((), ())),
        preferred_element_type=jnp.float32,
    )


def _matmul(x, w):
    m, k = x.shape
    n, _ = w.shape
    return pl.pallas_call(
        _matmul_kernel,
        out_shape=jax.ShapeDtypeStruct((m, n), jnp.float32),
    )(x, w)


def _attn_kernel(q_ref, k_ref, v_ref, o_ref, *, base, ext):
    iq = base + pl.program_id(1)
    q = q_ref[0]
    k = k_ref[0]
    v = v_ref[0]
    s = jax.lax.dot_general(
        q, k, dimension_numbers=(((1,), (1,)), ((), ())),
        preferred_element_type=jnp.float32,
    ) * _SCALE
    row = iq * _BQ + jax.lax.broadcasted_iota(jnp.int32, (_BQ, ext), 0)
    col = jax.lax.broadcasted_iota(jnp.int32, (_BQ, ext), 1)
    s = jnp.where(col <= row, s, -jnp.inf)
    m = jnp.max(s, axis=1, keepdims=True)
    p = jnp.exp(s - m)
    l = jnp.sum(p, axis=1, keepdims=True)
    o = jnp.dot(p.astype(jnp.bfloat16), v,
                preferred_element_type=jnp.float32) / l
    o_ref[0] = o


def _attention(q, k, v):
    outs = []
    for g in range(_T // (2 * _BQ)):
        ext = (2 * g + 2) * _BQ
        out_g = pl.pallas_call(
            functools.partial(_attn_kernel, base=2 * g, ext=ext),
            grid=(_NH, 2),
            in_specs=[
                pl.BlockSpec((1, _BQ, _HD), lambda h, i, g=g: (h, 2 * g + i, 0)),
                pl.BlockSpec((1, ext, _HD), lambda h, i: (h, 0, 0)),
                pl.BlockSpec((1, ext, _HD), lambda h, i: (h, 0, 0)),
            ],
            out_specs=pl.BlockSpec((1, _BQ, _HD), lambda h, i: (h, i, 0)),
            out_shape=jax.ShapeDtypeStruct((_NH, 2 * _BQ, _HD), jnp.float32),
        )(q, k, v)
        outs.append(out_g)
    return jnp.concatenate(outs, axis=1)


def kernel(x, W_attn, W_proj):
    b, t, c = x.shape
    x2 = x.reshape(t, c).astype(jnp.bfloat16)
    qkv = _matmul(x2, W_attn.astype(jnp.bfloat16))
    q, k, v = jnp.split(qkv.astype(jnp.bfloat16), 3, axis=1)
    q = q.reshape(t, _NH, _HD).transpose(1, 0, 2)
    k = k.reshape(t, _NH, _HD).transpose(1, 0, 2)
    v = v.reshape(t, _NH, _HD).transpose(1, 0, 2)
    y = _attention(q, k, v)
    y = y.transpose(1, 0, 2).reshape(t, c).astype(jnp.bfloat16)
    out = _matmul(y, W_proj.astype(jnp.bfloat16))
    return out.reshape(b, t, c)


# speedup vs baseline: 1.2839x; 1.2839x over previous
import functools

import jax
import jax.numpy as jnp
from jax.experimental import pallas as pl

_B, _T, _C, _NH = 1, 2048, 768, 32
_HD = _C // _NH
_BQ = 256
_SCALE = 1.0 / (_HD ** 0.5)


def _matmul_kernel(x_ref, w_ref, o_ref):
    o_ref[...] = jax.lax.dot_general(
        x_ref[...], w_ref[...],
        dimension_numbers=(((1,), (1,)), ((), ())),
        preferred_element_type=jnp.float32,
    )


def _matmul(x, w):
    m, k = x.shape
    n, _ = w.shape
    return pl.pallas_call(
        _matmul_kernel,
        out_shape=jax.ShapeDtypeStruct((m, n), jnp.float32),
    )(x, w)


def _attn_kernel(q_ref, kv_ref, o_ref, *, base, ext):
    iq = base + pl.program_id(0)
    row = iq * _BQ + jax.lax.broadcasted_iota(jnp.int32, (_BQ, ext), 0)
    col = jax.lax.broadcasted_iota(jnp.int32, (_BQ, ext), 1)
    keep = col <= row
    for h in range(_NH):
        q = q_ref[:, h * _HD:(h + 1) * _HD]
        k = kv_ref[:, _C + h * _HD:_C + (h + 1) * _HD]
        v = kv_ref[:, 2 * _C + h * _HD:2 * _C + (h + 1) * _HD]
        s = jax.lax.dot_general(
            q, k, dimension_numbers=(((1,), (1,)), ((), ())),
            preferred_element_type=jnp.float32,
        ) * _SCALE
        s = jnp.where(keep, s, -jnp.inf)
        m = jnp.max(s, axis=1, keepdims=True)
        p = jnp.exp(s - m)
        l = jnp.sum(p, axis=1, keepdims=True)
        o = jnp.dot(p.astype(jnp.bfloat16), v,
                    preferred_element_type=jnp.float32) / l
        o_ref[:, h * _HD:(h + 1) * _HD] = o


def _attention(qkv):
    outs = []
    for g in range(_T // (2 * _BQ)):
        ext = (2 * g + 2) * _BQ
        out_g = pl.pallas_call(
            functools.partial(_attn_kernel, base=2 * g, ext=ext),
            grid=(2,),
            in_specs=[
                pl.BlockSpec((_BQ, 3 * _C), lambda i, g=g: (2 * g + i, 0)),
                pl.BlockSpec((ext, 3 * _C), lambda i: (0, 0)),
            ],
            out_specs=pl.BlockSpec((_BQ, _C), lambda i: (i, 0)),
            out_shape=jax.ShapeDtypeStruct((2 * _BQ, _C), jnp.float32),
        )(qkv, qkv)
        outs.append(out_g)
    return jnp.concatenate(outs, axis=0)


def kernel(x, W_attn, W_proj):
    b, t, c = x.shape
    x2 = x.reshape(t, c)
    qkv = _matmul(x2, W_attn)
    y = _attention(qkv.astype(jnp.bfloat16))
    out = _matmul(y, W_proj)
    return out.reshape(b, t, c)
